# baseline (speedup 1.0000x reference)
import functools

import jax
import jax.numpy as jnp
from jax import lax
from jax.experimental import pallas as pl
from jax.experimental.pallas import tpu as pltpu
from jax.experimental.pallas import tpu_sc as plsc

N_NODES = 10000
N_EDGES = 160000
D_IN = 256
D_HID = 512
N_CLASSES = 10
N_GRAPHS = 128

NC = 2
NS = 16
LANES = 16

EPT = N_EDGES // NS
B = 112
NB = 90
SLAB = 15
NSLAB = NB // SLAB
NRING = 3
EPT_PAD = NB * B
TAB_ROWS = 10240
ZROWS = 8
WB = 632

NW = NC * NS
EPW = N_EDGES // NW


def _mesh():
    return plsc.VectorSubcoreMesh(
        core_axis_name="c", subcore_axis_name="s", num_cores=NC, num_subcores=NS
    )


@functools.partial(
    pl.kernel,
    out_type=jax.ShapeDtypeStruct((NW, N_NODES), jnp.float32),
    mesh=_mesh(),
    scratch_types=[
        pltpu.VMEM((EPW + 16,), jnp.int32),
        pltpu.VMEM((N_NODES,), jnp.float32),
    ],
    compiler_params=pltpu.CompilerParams(needs_layout_passes=False),
)
def _sc_deg(dst_hbm, out_hbm, ids_v, tab_v):
    c = lax.axis_index("c")
    s = lax.axis_index("s")
    wid = c * NS + s
    zeros16 = jnp.zeros((LANES,), jnp.float32)

    def zero_body(i, _):
        tab_v[pl.ds(i * LANES, LANES)] = zeros16
        return 0

    lax.fori_loop(0, N_NODES // LANES, zero_body, 0)

    pltpu.sync_copy(dst_hbm.at[pl.ds(wid * EPW, EPW)], ids_v.at[pl.ds(0, EPW)])

    lane = lax.iota(jnp.int32, LANES)
    ones16 = jnp.ones((LANES,), jnp.float32)

    def body(i, _):
        idx = ids_v[pl.ds(i * LANES, LANES)]
        mask = (i * LANES + lane) < EPW
        idx = jnp.where(mask, idx, 0)
        plsc.addupdate_scatter(tab_v, [idx], ones16, mask=mask)
        return 0

    lax.fori_loop(0, (EPW + LANES - 1) // LANES, body, 0)
    pltpu.sync_copy(tab_v, out_hbm.at[wid])


def _make_sc_agg(nchunk):
    npc = nchunk // NC

    def body(*refs):
        y_refs = refs[:nchunk]
        src_hbm = refs[nchunk]
        dst_hbm = refs[nchunk + 1]
        agg_refs = refs[nchunk + 2 : 2 * nchunk + 2]
        (table, src_v, dst_v, b0, b1, b2, zbuf,
         g0, g1, g2, s0, s1, s2) = refs[2 * nchunk + 2 :]
        bufs = (b0, b1, b2)
        gsems = (g0, g1, g2)
        ssems = (s0, s1, s2)

        c = lax.axis_index("c")
        s = lax.axis_index("s")

        z16 = jnp.zeros((LANES,), jnp.float32)
        for i in range(ZROWS):
            for j in range(128 // LANES):
                zbuf[i, pl.ds(j * LANES, LANES)] = z16

        rows_per_tile = TAB_ROWS // NS

        def zero_tab(base):
            def zb(j, _):
                pltpu.sync_copy(zbuf, table.at[pl.ds(base + j * ZROWS, ZROWS)])
                return 0

            lax.fori_loop(0, rows_per_tile // ZROWS, zb, 0)

        zero_tab(s * rows_per_tile)
        plsc.subcore_barrier()

        def process(y_ref, agg_ref, rezero):
            def gth(lb, r):
                pltpu.async_copy(
                    y_ref.at[src_v.at[pl.ds(lb * B, B)]], bufs[r], gsems[r]
                )

            def gwait(lb, r):
                pltpu.make_async_copy(
                    y_ref.at[src_v.at[pl.ds(lb * B, B)]], bufs[r], gsems[r]
                ).wait()

            def sct(lb, r):
                pltpu.async_copy(bufs[r], table.at[dst_v.at[lb]], ssems[r], add=True)

            def swait(lb, r):
                pltpu.make_async_copy(
                    bufs[r], table.at[dst_v.at[lb]], ssems[r]
                ).wait()

            def slab(t, _):
                pltpu.sync_copy(src_hbm.at[s, pl.ds(t * SLAB * B, SLAB * B)], src_v)
                pltpu.sync_copy(dst_hbm.at[s, pl.ds(t * SLAB, SLAB)], dst_v)

                def grp(g, _):
                    for r in range(NRING):
                        k = g * NRING + r

                        @pl.when(g >= 1)
                        def _(k=k, r=r):
                            swait(k - NRING, r)

                        gth(k, r)
                        if r == 0:

                            @pl.when(g >= 1)
                            def _(k=k):
                                gwait(k - 1, NRING - 1)
                                sct(k - 1, NRING - 1)

                        else:
                            gwait(k - 1, r - 1)
                            sct(k - 1, r - 1)
                    return 0

                lax.fori_loop(0, SLAB // NRING, grp, 0)
                last = SLAB - 1
                gwait(last, NRING - 1)
                sct(last, NRING - 1)
                for r in range(NRING):
                    swait(SLAB - NRING + r, r)
                return 0

            lax.fori_loop(0, NSLAB, slab, 0)
            plsc.subcore_barrier()

            @pl.when(s < NS - 1)
            def _():
                pltpu.sync_copy(
                    table.at[pl.ds(s * WB, WB)], agg_ref.at[pl.ds(s * WB, WB)]
                )

            @pl.when(s == NS - 1)
            def _():
                last = N_NODES - (NS - 1) * WB
                pltpu.sync_copy(
                    table.at[pl.ds((NS - 1) * WB, last)],
                    agg_ref.at[pl.ds((NS - 1) * WB, last)],
                )

            if rezero:
                plsc.subcore_barrier()
                zero_tab(s * rows_per_tile)
                plsc.subcore_barrier()

        for cc in range(NC):

            @pl.when(c == cc)
            def _(cc=cc):
                for k in range(npc):
                    ch = cc * npc + k
                    process(y_refs[ch], agg_refs[ch], rezero=(k < npc - 1))

    out_t = [jax.ShapeDtypeStruct((N_NODES, 128), jnp.float32)] * nchunk
    return pl.kernel(
        body,
        out_type=out_t,
        mesh=_mesh(),
        scratch_types=[
            pltpu.VMEM_SHARED((TAB_ROWS, 128), jnp.float32),
            pltpu.VMEM((SLAB * B,), jnp.int32),
            pltpu.VMEM((SLAB, B), jnp.int32),
            pltpu.VMEM((B, 128), jnp.float32),
            pltpu.VMEM((B, 128), jnp.float32),
            pltpu.VMEM((B, 128), jnp.float32),
            pltpu.VMEM((ZROWS, 128), jnp.float32),
            pltpu.SemaphoreType.DMA,
            pltpu.SemaphoreType.DMA,
            pltpu.SemaphoreType.DMA,
            pltpu.SemaphoreType.DMA,
            pltpu.SemaphoreType.DMA,
            pltpu.SemaphoreType.DMA,
        ],
        compiler_params=pltpu.CompilerParams(use_tc_tiling_on_sc=False),
    )


_sc_agg2 = _make_sc_agg(2)
_sc_agg4 = _make_sc_agg(4)


_R = 1000


def _ck(i):
    return pl.BlockSpec((_R, 128), lambda i: (i, 0))


_CHUNK_SPEC = pl.BlockSpec((_R, 128), lambda i: (i, 0))
_CHUNK_OUT = jax.ShapeDtypeStruct((N_NODES, 128), jnp.float32)


def _tc_prep_body(p_ref, x_ref, y0_ref, y1_ref, dinv_ref):
    p = p_ref[...]
    ones = jnp.ones((NW, 1), jnp.float32)
    deg = jnp.dot(p, ones, preferred_element_type=jnp.float32)
    dinv = lax.rsqrt(deg + 1.0)
    dinv_ref[...] = dinv
    x = x_ref[...]
    y0_ref[...] = x[:, 0:128] * dinv
    y1_ref[...] = x[:, 128:256] * dinv


def _tc_prep(partials_t, x):
    return pl.pallas_call(
        _tc_prep_body,
        grid=(N_NODES // _R,),
        in_specs=[
            pl.BlockSpec((_R, NW), lambda i: (i, 0)),
            pl.BlockSpec((_R, D_IN), lambda i: (i, 0)),
        ],
        out_specs=[_CHUNK_SPEC, _CHUNK_SPEC, pl.BlockSpec((_R, 1), lambda i: (i, 0))],
        out_shape=[_CHUNK_OUT, _CHUNK_OUT,
                   jax.ShapeDtypeStruct((N_NODES, 1), jnp.float32)],
    )(partials_t, x)


def _make_tc_layer(nin):
    nout = D_HID // 128

    def body(*refs):
        y = refs[:nin]
        agg = refs[nin : 2 * nin]
        dinv_ref, w_ref, b_ref = refs[2 * nin : 2 * nin + 3]
        outs = refs[2 * nin + 3 :]
        dinv = dinv_ref[...]
        z = jnp.concatenate(
            [(y[c][...] + agg[c][...]) for c in range(nin)], axis=1
        ) * dinv
        h = (
            jnp.dot(z.astype(jnp.bfloat16), w_ref[...],
                    preferred_element_type=jnp.float32)
            + b_ref[...]
        )
        h = jnp.maximum(h, 0.0) * dinv
        for c in range(nout):
            outs[c][...] = h[:, c * 128 : (c + 1) * 128]

    din = nin * 128

    def call(*args):
        return pl.pallas_call(
            body,
            grid=(N_NODES // _R,),
            in_specs=(
                [_CHUNK_SPEC] * (2 * nin)
                + [
                    pl.BlockSpec((_R, 1), lambda i: (i, 0)),
                    pl.BlockSpec((din, D_HID), lambda i: (0, 0)),
                    pl.BlockSpec((1, D_HID), lambda i: (0, 0)),
                ]
            ),
            out_specs=[_CHUNK_SPEC] * nout,
            out_shape=[_CHUNK_OUT] * nout,
        )(*args)

    return call


_tc_layer2 = _make_tc_layer(2)
_tc_layer4 = _make_tc_layer(4)

_NHID = D_HID // 128


def _tc_final_body(*refs):
    y = refs[:_NHID]
    agg = refs[_NHID : 2 * _NHID]
    (dinv_ref, bat_ref, w3_ref, b3_ref, wl_ref, bl_ref,
     out_ref, hp_ref, acc, cnt) = refs[2 * _NHID :]
    i = pl.program_id(0)

    @pl.when(i == 0)
    def _():
        acc[...] = jnp.zeros_like(acc)
        cnt[...] = jnp.zeros_like(cnt)

    z = jnp.concatenate(
        [(y[c][...] + agg[c][...]) for c in range(_NHID)], axis=1
    ) * dinv_ref[...]
    gids = lax.broadcasted_iota(jnp.int32, (1, N_GRAPHS), 1)
    p = (bat_ref[...] == gids).astype(jnp.bfloat16)
    acc[...] += lax.dot_general(
        p, z.astype(jnp.bfloat16), (((0,), (0,)), ((), ())),
        preferred_element_type=jnp.float32,
    )
    ones = jnp.ones((_R, 1), jnp.bfloat16)
    cnt[...] += lax.dot_general(
        p, ones, (((0,), (0,)), ((), ())), preferred_element_type=jnp.float32
    )

    @pl.when(i == pl.num_programs(0) - 1)
    def _():
        zp = acc[...] / jnp.maximum(cnt[...], 1.0)
        hp = jnp.dot(zp, w3_ref[...], preferred_element_type=jnp.float32) + b3_ref[...]
        hp_ref[...] = hp
        out_ref[...] = (
            jnp.dot(hp, wl_ref[...], preferred_element_type=jnp.float32) + bl_ref[...]
        )


def _tc_final(*args):
    return pl.pallas_call(
        _tc_final_body,
        grid=(N_NODES // _R,),
        in_specs=(
            [_CHUNK_SPEC] * (2 * _NHID)
            + [
                pl.BlockSpec((_R, 1), lambda i: (i, 0)),
                pl.BlockSpec((_R, 1), lambda i: (i, 0)),
                pl.BlockSpec((D_HID, D_HID), lambda i: (0, 0)),
                pl.BlockSpec((1, D_HID), lambda i: (0, 0)),
                pl.BlockSpec((D_HID, N_CLASSES), lambda i: (0, 0)),
                pl.BlockSpec((1, N_CLASSES), lambda i: (0, 0)),
            ]
        ),
        out_specs=[
            pl.BlockSpec((N_GRAPHS, N_CLASSES), lambda i: (0, 0)),
            pl.BlockSpec((N_GRAPHS, D_HID), lambda i: (0, 0)),
        ],
        out_shape=[
            jax.ShapeDtypeStruct((N_GRAPHS, N_CLASSES), jnp.float32),
            jax.ShapeDtypeStruct((N_GRAPHS, D_HID), jnp.float32),
        ],
        scratch_shapes=[
            pltpu.VMEM((N_GRAPHS, D_HID), jnp.float32),
            pltpu.VMEM((N_GRAPHS, 1), jnp.float32),
        ],
    )(*args)


def kernel(x, edge_index, batch, W1, b1, W2, b2, W3, b3, Wl, bl):
    src = edge_index[0].astype(jnp.int32)
    dst = edge_index[1].astype(jnp.int32)

    pad = EPT_PAD - EPT
    src_t = jnp.concatenate(
        [src.reshape(NS, EPT), jnp.zeros((NS, pad), jnp.int32)], axis=1
    )
    dst_t = jnp.concatenate(
        [dst.reshape(NS, EPT), jnp.full((NS, pad), N_NODES, jnp.int32)], axis=1
    ).reshape(NS, NB, B)

    partials = _sc_deg(dst)
    y00, y01, dinv = _tc_prep(partials.T, x)

    agg0 = _sc_agg2(y00, y01, src_t, dst_t)
    y1 = _tc_layer2(y00, y01, agg0[0], agg0[1], dinv,
                    W1.astype(jnp.bfloat16), b1.reshape(1, -1))

    agg1 = _sc_agg4(*y1, src_t, dst_t)
    y2 = _tc_layer4(*y1, *agg1, dinv,
                    W2.astype(jnp.bfloat16), b2.reshape(1, -1))

    agg2 = _sc_agg4(*y2, src_t, dst_t)
    out, hp = _tc_final(
        *y2, *agg2, dinv, batch.reshape(-1, 1).astype(jnp.int32),
        W3, b3.reshape(1, -1), Wl, bl.reshape(1, -1),
    )
    return (out, hp)


# speedup vs baseline: 1.2394x; 1.2394x over previous
import functools

import jax
import jax.numpy as jnp
from jax import lax
from jax.experimental import pallas as pl
from jax.experimental.pallas import tpu as pltpu
from jax.experimental.pallas import tpu_sc as plsc

N_NODES = 10000
N_EDGES = 160000
D_IN = 256
D_HID = 512
N_CLASSES = 10
N_GRAPHS = 128

NC = 2
NS = 16
LANES = 16

EPT = N_EDGES // NS
B = 112
NB = 90
SLAB = 15
NSLAB = NB // SLAB
NRING = 3
EPT_PAD = NB * B
TAB_ROWS = 10240
ZROWS = 8
WB = 632

NW = NC * NS
EPW = N_EDGES // NW


def _mesh():
    return plsc.VectorSubcoreMesh(
        core_axis_name="c", subcore_axis_name="s", num_cores=NC, num_subcores=NS
    )


@functools.partial(
    pl.kernel,
    out_type=jax.ShapeDtypeStruct((NW, N_NODES), jnp.float32),
    mesh=_mesh(),
    scratch_types=[
        pltpu.VMEM((EPW + 16,), jnp.int32),
        pltpu.VMEM((N_NODES,), jnp.float32),
    ],
    compiler_params=pltpu.CompilerParams(needs_layout_passes=False),
)
def _sc_deg(dst_hbm, out_hbm, ids_v, tab_v):
    c = lax.axis_index("c")
    s = lax.axis_index("s")
    wid = c * NS + s
    zeros16 = jnp.zeros((LANES,), jnp.float32)

    def zero_body(i, _):
        tab_v[pl.ds(i * LANES, LANES)] = zeros16
        return 0

    lax.fori_loop(0, N_NODES // LANES, zero_body, 0)

    pltpu.sync_copy(dst_hbm.at[pl.ds(wid * EPW, EPW)], ids_v.at[pl.ds(0, EPW)])

    lane = lax.iota(jnp.int32, LANES)
    ones16 = jnp.ones((LANES,), jnp.float32)

    def body(i, _):
        idx = ids_v[pl.ds(i * LANES, LANES)]
        mask = (i * LANES + lane) < EPW
        idx = jnp.where(mask, idx, 0)
        plsc.addupdate_scatter(tab_v, [idx], ones16, mask=mask)
        return 0

    lax.fori_loop(0, (EPW + LANES - 1) // LANES, body, 0)
    pltpu.sync_copy(tab_v, out_hbm.at[wid])


def _make_sc_agg(nchunk):
    npc = nchunk // NC

    def body(*refs):
        y_refs = refs[:nchunk]
        src_hbm = refs[nchunk]
        dst_hbm = refs[nchunk + 1]
        agg_refs = refs[nchunk + 2 : 2 * nchunk + 2]
        (table, src_v, dst_v, b0, b1, b2, zbuf,
         g0, g1, g2, s0, s1, s2) = refs[2 * nchunk + 2 :]
        bufs = (b0, b1, b2)
        gsems = (g0, g1, g2)
        ssems = (s0, s1, s2)

        c = lax.axis_index("c")
        s = lax.axis_index("s")

        z32 = jnp.zeros((2 * LANES,), jnp.bfloat16)
        for i in range(ZROWS):
            for j in range(128 // (2 * LANES)):
                zbuf[i, pl.ds(j * 2 * LANES, 2 * LANES)] = z32

        rows_per_tile = TAB_ROWS // NS

        def zero_tab(base):
            def zb(j, _):
                pltpu.sync_copy(zbuf, table.at[pl.ds(base + j * ZROWS, ZROWS)])
                return 0

            lax.fori_loop(0, rows_per_tile // ZROWS, zb, 0)

        zero_tab(s * rows_per_tile)
        plsc.subcore_barrier()

        def process(y_ref, agg_ref, rezero):
            def gth(lb, r):
                pltpu.async_copy(
                    y_ref.at[src_v.at[pl.ds(lb * B, B)]], bufs[r], gsems[r]
                )

            def gwait(lb, r):
                pltpu.make_async_copy(
                    y_ref.at[src_v.at[pl.ds(lb * B, B)]], bufs[r], gsems[r]
                ).wait()

            def sct(lb, r):
                pltpu.async_copy(bufs[r], table.at[dst_v.at[lb]], ssems[r], add=True)

            def swait(lb, r):
                pltpu.make_async_copy(
                    bufs[r], table.at[dst_v.at[lb]], ssems[r]
                ).wait()

            def slab(t, _):
                pltpu.sync_copy(src_hbm.at[s, pl.ds(t * SLAB * B, SLAB * B)], src_v)
                pltpu.sync_copy(dst_hbm.at[s, pl.ds(t * SLAB, SLAB)], dst_v)

                def grp(g, _):
                    for r in range(NRING):
                        k = g * NRING + r

                        @pl.when(g >= 1)
                        def _(k=k, r=r):
                            swait(k - NRING, r)

                        gth(k, r)
                        if r == 0:

                            @pl.when(g >= 1)
                            def _(k=k):
                                gwait(k - 1, NRING - 1)
                                sct(k - 1, NRING - 1)

                        else:
                            gwait(k - 1, r - 1)
                            sct(k - 1, r - 1)
                    return 0

                lax.fori_loop(0, SLAB // NRING, grp, 0)
                last = SLAB - 1
                gwait(last, NRING - 1)
                sct(last, NRING - 1)
                for r in range(NRING):
                    swait(SLAB - NRING + r, r)
                return 0

            lax.fori_loop(0, NSLAB, slab, 0)
            plsc.subcore_barrier()

            @pl.when(s < NS - 1)
            def _():
                pltpu.sync_copy(
                    table.at[pl.ds(s * WB, WB)], agg_ref.at[pl.ds(s * WB, WB)]
                )

            @pl.when(s == NS - 1)
            def _():
                last = N_NODES - (NS - 1) * WB
                pltpu.sync_copy(
                    table.at[pl.ds((NS - 1) * WB, last)],
                    agg_ref.at[pl.ds((NS - 1) * WB, last)],
                )

            if rezero:
                plsc.subcore_barrier()
                zero_tab(s * rows_per_tile)
                plsc.subcore_barrier()

        for cc in range(NC):

            @pl.when(c == cc)
            def _(cc=cc):
                for k in range(npc):
                    ch = cc * npc + k
                    process(y_refs[ch], agg_refs[ch], rezero=(k < npc - 1))

    out_t = [jax.ShapeDtypeStruct((N_NODES, 128), jnp.bfloat16)] * nchunk
    return pl.kernel(
        body,
        out_type=out_t,
        mesh=_mesh(),
        scratch_types=[
            pltpu.VMEM_SHARED((TAB_ROWS, 128), jnp.bfloat16),
            pltpu.VMEM((SLAB * B,), jnp.int32),
            pltpu.VMEM((SLAB, B), jnp.int32),
            pltpu.VMEM((B, 128), jnp.bfloat16),
            pltpu.VMEM((B, 128), jnp.bfloat16),
            pltpu.VMEM((B, 128), jnp.bfloat16),
            pltpu.VMEM((ZROWS, 128), jnp.bfloat16),
            pltpu.SemaphoreType.DMA,
            pltpu.SemaphoreType.DMA,
            pltpu.SemaphoreType.DMA,
            pltpu.SemaphoreType.DMA,
            pltpu.SemaphoreType.DMA,
            pltpu.SemaphoreType.DMA,
        ],
        compiler_params=pltpu.CompilerParams(use_tc_tiling_on_sc=False),
    )


_sc_agg2 = _make_sc_agg(2)
_sc_agg4 = _make_sc_agg(4)


_R = 1000


def _ck(i):
    return pl.BlockSpec((_R, 128), lambda i: (i, 0))


_CHUNK_SPEC = pl.BlockSpec((_R, 128), lambda i: (i, 0))
_CHUNK_OUT = jax.ShapeDtypeStruct((N_NODES, 128), jnp.bfloat16)


def _tc_prep_body(p_ref, x_ref, y0_ref, y1_ref, dinv_ref):
    p = p_ref[...]
    ones = jnp.ones((NW, 1), jnp.float32)
    deg = jnp.dot(p, ones, preferred_element_type=jnp.float32)
    dinv = lax.rsqrt(deg + 1.0)
    dinv_ref[...] = dinv
    x = x_ref[...]
    y0_ref[...] = (x[:, 0:128] * dinv).astype(jnp.bfloat16)
    y1_ref[...] = (x[:, 128:256] * dinv).astype(jnp.bfloat16)


def _tc_prep(partials_t, x):
    return pl.pallas_call(
        _tc_prep_body,
        grid=(N_NODES // _R,),
        in_specs=[
            pl.BlockSpec((_R, NW), lambda i: (i, 0)),
            pl.BlockSpec((_R, D_IN), lambda i: (i, 0)),
        ],
        out_specs=[_CHUNK_SPEC, _CHUNK_SPEC, pl.BlockSpec((_R, 1), lambda i: (i, 0))],
        out_shape=[_CHUNK_OUT, _CHUNK_OUT,
                   jax.ShapeDtypeStruct((N_NODES, 1), jnp.float32)],
    )(partials_t, x)


def _make_tc_layer(nin):
    nout = D_HID // 128

    def body(*refs):
        y = refs[:nin]
        agg = refs[nin : 2 * nin]
        dinv_ref, w_ref, b_ref = refs[2 * nin : 2 * nin + 3]
        outs = refs[2 * nin + 3 :]
        dinv = dinv_ref[...]
        z = jnp.concatenate(
            [(y[c][...].astype(jnp.float32) + agg[c][...].astype(jnp.float32))
             for c in range(nin)], axis=1
        ) * dinv
        h = (
            jnp.dot(z.astype(jnp.bfloat16), w_ref[...],
                    preferred_element_type=jnp.float32)
            + b_ref[...]
        )
        h = jnp.maximum(h, 0.0) * dinv
        for c in range(nout):
            outs[c][...] = h[:, c * 128 : (c + 1) * 128].astype(jnp.bfloat16)

    din = nin * 128

    def call(*args):
        return pl.pallas_call(
            body,
            grid=(N_NODES // _R,),
            in_specs=(
                [_CHUNK_SPEC] * (2 * nin)
                + [
                    pl.BlockSpec((_R, 1), lambda i: (i, 0)),
                    pl.BlockSpec((din, D_HID), lambda i: (0, 0)),
                    pl.BlockSpec((1, D_HID), lambda i: (0, 0)),
                ]
            ),
            out_specs=[_CHUNK_SPEC] * nout,
            out_shape=[_CHUNK_OUT] * nout,
        )(*args)

    return call


_tc_layer2 = _make_tc_layer(2)
_tc_layer4 = _make_tc_layer(4)

_NHID = D_HID // 128


def _tc_final_body(*refs):
    y = refs[:_NHID]
    agg = refs[_NHID : 2 * _NHID]
    (dinv_ref, bat_ref, w3_ref, b3_ref, wl_ref, bl_ref,
     out_ref, hp_ref, acc, cnt) = refs[2 * _NHID :]
    i = pl.program_id(0)

    @pl.when(i == 0)
    def _():
        acc[...] = jnp.zeros_like(acc)
        cnt[...] = jnp.zeros_like(cnt)

    z = jnp.concatenate(
        [(y[c][...].astype(jnp.float32) + agg[c][...].astype(jnp.float32))
         for c in range(_NHID)], axis=1
    ) * dinv_ref[...]
    gids = lax.broadcasted_iota(jnp.int32, (1, N_GRAPHS), 1)
    p = (bat_ref[...] == gids).astype(jnp.bfloat16)
    acc[...] += lax.dot_general(
        p, z.astype(jnp.bfloat16), (((0,), (0,)), ((), ())),
        preferred_element_type=jnp.float32,
    )
    ones = jnp.ones((_R, 1), jnp.bfloat16)
    cnt[...] += lax.dot_general(
        p, ones, (((0,), (0,)), ((), ())), preferred_element_type=jnp.float32
    )

    @pl.when(i == pl.num_programs(0) - 1)
    def _():
        zp = acc[...] / jnp.maximum(cnt[...], 1.0)
        hp = jnp.dot(zp, w3_ref[...], preferred_element_type=jnp.float32) + b3_ref[...]
        hp_ref[...] = hp
        out_ref[...] = (
            jnp.dot(hp, wl_ref[...], preferred_element_type=jnp.float32) + bl_ref[...]
        )


def _tc_final(*args):
    return pl.pallas_call(
        _tc_final_body,
        grid=(N_NODES // _R,),
        in_specs=(
            [_CHUNK_SPEC] * (2 * _NHID)
            + [
                pl.BlockSpec((_R, 1), lambda i: (i, 0)),
                pl.BlockSpec((_R, 1), lambda i: (i, 0)),
                pl.BlockSpec((D_HID, D_HID), lambda i: (0, 0)),
                pl.BlockSpec((1, D_HID), lambda i: (0, 0)),
                pl.BlockSpec((D_HID, N_CLASSES), lambda i: (0, 0)),
                pl.BlockSpec((1, N_CLASSES), lambda i: (0, 0)),
            ]
        ),
        out_specs=[
            pl.BlockSpec((N_GRAPHS, N_CLASSES), lambda i: (0, 0)),
            pl.BlockSpec((N_GRAPHS, D_HID), lambda i: (0, 0)),
        ],
        out_shape=[
            jax.ShapeDtypeStruct((N_GRAPHS, N_CLASSES), jnp.float32),
            jax.ShapeDtypeStruct((N_GRAPHS, D_HID), jnp.float32),
        ],
        scratch_shapes=[
            pltpu.VMEM((N_GRAPHS, D_HID), jnp.float32),
            pltpu.VMEM((N_GRAPHS, 1), jnp.float32),
        ],
    )(*args)


def kernel(x, edge_index, batch, W1, b1, W2, b2, W3, b3, Wl, bl):
    src = edge_index[0].astype(jnp.int32)
    dst = edge_index[1].astype(jnp.int32)

    pad = EPT_PAD - EPT
    src_t = jnp.concatenate(
        [src.reshape(NS, EPT), jnp.zeros((NS, pad), jnp.int32)], axis=1
    )
    dst_t = jnp.concatenate(
        [dst.reshape(NS, EPT), jnp.full((NS, pad), N_NODES, jnp.int32)], axis=1
    ).reshape(NS, NB, B)

    partials = _sc_deg(dst)
    y00, y01, dinv = _tc_prep(partials.T, x)

    agg0 = _sc_agg2(y00, y01, src_t, dst_t)
    y1 = _tc_layer2(y00, y01, agg0[0], agg0[1], dinv,
                    W1.astype(jnp.bfloat16), b1.reshape(1, -1))

    agg1 = _sc_agg4(*y1, src_t, dst_t)
    y2 = _tc_layer4(*y1, *agg1, dinv,
                    W2.astype(jnp.bfloat16), b2.reshape(1, -1))

    agg2 = _sc_agg4(*y2, src_t, dst_t)
    out, hp = _tc_final(
        *y2, *agg2, dinv, batch.reshape(-1, 1).astype(jnp.int32),
        W3, b3.reshape(1, -1), Wl, bl.reshape(1, -1),
    )
    return (out, hp)
